# Initial kernel scaffold; baseline (speedup 1.0000x reference)
import jax
import jax.numpy as jnp
from jax.experimental import pallas as pl


def kernel(text, words_per_sentence, table, W1, b1, W2, b2):
    raise NotImplementedError("write your pallas kernel here")


# speedup vs baseline: 7.3560x; 7.3560x over previous
import functools

import jax
import jax.numpy as jnp
from jax import lax
from jax.experimental import pallas as pl
from jax.experimental.pallas import tpu as pltpu
from jax.experimental.pallas import tpu_sc as plsc

VOCAB = 100000
EMB = 64
HIDDEN = 512
N_CLASSES = 100
BATCH = 4096
PAD_LEN = 200

_NC = 2
_NS = 16
_NW = _NC * _NS
_BPW = BATCH // _NW
_CH0 = 128
_CH1 = PAD_LEN - _CH0


def _gather_mean_body(text_hbm, table_hbm, out_hbm, idx_v, rows_v, out_v, sem):
    wid = lax.axis_index("s") * _NC + lax.axis_index("c")
    base = wid * _BPW

    def sent_body(s, carry):
        gs = base + s
        pltpu.sync_copy(text_hbm.at[gs], idx_v)
        cp0 = pltpu.async_copy(
            table_hbm.at[idx_v.at[pl.ds(0, _CH0)]], rows_v.at[pl.ds(0, _CH0)], sem
        )
        cp1 = pltpu.async_copy(
            table_hbm.at[idx_v.at[pl.ds(_CH0, _CH1)]],
            rows_v.at[pl.ds(_CH0, _CH1)],
            sem,
        )
        cp0.wait()
        cp1.wait()

        def tok(t, acc):
            return tuple(
                acc[c] + rows_v[t, pl.ds(16 * c, 16)] for c in range(EMB // 16)
            )

        acc = lax.fori_loop(
            0,
            PAD_LEN,
            tok,
            tuple(jnp.zeros((16,), jnp.float32) for _ in range(EMB // 16)),
        )
        scale = jnp.float32(1.0 / PAD_LEN)
        for c in range(EMB // 16):
            out_v[s, pl.ds(16 * c, 16)] = acc[c] * scale
        return carry

    lax.fori_loop(0, _BPW, sent_body, 0)
    pltpu.sync_copy(out_v, out_hbm.at[pl.ds(base, _BPW)])


def _gather_mean(text, table):
    mesh = plsc.VectorSubcoreMesh(
        core_axis_name="c", subcore_axis_name="s", num_cores=_NC, num_subcores=_NS
    )
    k = pl.kernel(
        _gather_mean_body,
        out_type=jax.ShapeDtypeStruct((BATCH, EMB), jnp.float32),
        mesh=mesh,
        compiler_params=pltpu.CompilerParams(use_tc_tiling_on_sc=False),
        scratch_types=[
            pltpu.VMEM((PAD_LEN,), jnp.int32),
            pltpu.VMEM((PAD_LEN, EMB), jnp.float32),
            pltpu.VMEM((_BPW, EMB), jnp.float32),
            pltpu.SemaphoreType.DMA,
        ],
    )
    return k(text, table)


_NC_PAD = 128


def _mlp_body(avg_ref, w1_ref, b1_ref, w2_ref, b2_ref, out_ref):
    h = (
        jnp.dot(avg_ref[:], w1_ref[:], preferred_element_type=jnp.float32)
        + b1_ref[:]
    )
    out_ref[:] = (
        jnp.dot(h, w2_ref[:], preferred_element_type=jnp.float32) + b2_ref[:]
    )


def _mlp(avg, W1, b1, W2, b2):
    blk = 1024
    W2p = jnp.pad(W2, ((0, 0), (0, _NC_PAD - N_CLASSES)))
    b2p = jnp.pad(b2, (0, _NC_PAD - N_CLASSES))
    out = pl.pallas_call(
        _mlp_body,
        grid=(BATCH // blk,),
        in_specs=[
            pl.BlockSpec((blk, EMB), lambda i: (i, 0)),
            pl.BlockSpec((EMB, HIDDEN), lambda i: (0, 0)),
            pl.BlockSpec((1, HIDDEN), lambda i: (0, 0)),
            pl.BlockSpec((HIDDEN, _NC_PAD), lambda i: (0, 0)),
            pl.BlockSpec((1, _NC_PAD), lambda i: (0, 0)),
        ],
        out_specs=pl.BlockSpec((blk, _NC_PAD), lambda i: (i, 0)),
        out_shape=jax.ShapeDtypeStruct((BATCH, _NC_PAD), jnp.float32),
    )(avg, W1, b1.reshape(1, HIDDEN), W2p, b2p.reshape(1, _NC_PAD))
    return out[:, :N_CLASSES]


def kernel(text, words_per_sentence, table, W1, b1, W2, b2):
    avg = _gather_mean(text, table)
    return _mlp(avg, W1, b1, W2, b2)


# speedup vs baseline: 13.9465x; 1.8959x over previous
import functools

import jax
import jax.numpy as jnp
from jax import lax
from jax.experimental import pallas as pl
from jax.experimental.pallas import tpu as pltpu
from jax.experimental.pallas import tpu_sc as plsc

VOCAB = 100000
EMB = 64
HIDDEN = 512
N_CLASSES = 100
BATCH = 4096
PAD_LEN = 200

_NC = 2
_NS = 16
_NW = _NC * _NS
_BPW = BATCH // _NW
_CH0 = 128
_CH1 = PAD_LEN - _CH0


def _gather_mean_body(text_hbm, table_hbm, out_hbm, idx_v, rows_v, out_v, sem0, sem1):
    wid = lax.axis_index("s") * _NC + lax.axis_index("c")
    base = wid * _BPW
    pltpu.sync_copy(text_hbm.at[pl.ds(base, _BPW)], idx_v)
    sems = (sem0, sem1)

    def _copies(s, b):
        sc = jnp.minimum(s, _BPW - 1)
        c0 = pltpu.make_async_copy(
            table_hbm.at[idx_v.at[sc, pl.ds(0, _CH0)]],
            rows_v.at[b, pl.ds(0, _CH0)],
            sems[b],
        )
        c1 = pltpu.make_async_copy(
            table_hbm.at[idx_v.at[sc, pl.ds(_CH0, _CH1)]],
            rows_v.at[b, pl.ds(_CH0, _CH1)],
            sems[b],
        )
        return c0, c1

    def _issue(s, b):
        for c in _copies(s, b):
            c.start()

    def _drain(s, b):
        for c in _copies(s, b):
            c.wait()

    scale = jnp.float32(1.0 / PAD_LEN)
    zeros = tuple(jnp.zeros((16,), jnp.float32) for _ in range(EMB // 16))

    def _accum(s, b):
        def tok2(t, acc):
            t0 = 2 * t
            acc = tuple(
                acc[c] + rows_v[b, t0, pl.ds(16 * c, 16)] for c in range(EMB // 16)
            )
            return tuple(
                acc[c] + rows_v[b, t0 + 1, pl.ds(16 * c, 16)]
                for c in range(EMB // 16)
            )

        acc = lax.fori_loop(0, PAD_LEN // 2, tok2, zeros)
        for c in range(EMB // 16):
            out_v[s, pl.ds(16 * c, 16)] = acc[c] * scale

    _issue(0, 0)
    _issue(1, 1)

    def pair_body(i, carry):
        s = 2 * i
        _drain(s, 0)
        _accum(s, 0)
        _issue(s + 2, 0)
        _drain(s + 1, 1)
        _accum(s + 1, 1)
        _issue(s + 3, 1)
        return carry

    lax.fori_loop(0, _BPW // 2, pair_body, 0)
    _drain(_BPW, 0)
    _drain(_BPW + 1, 1)
    pltpu.sync_copy(out_v, out_hbm.at[pl.ds(base, _BPW)])


def _gather_mean(text, table):
    mesh = plsc.VectorSubcoreMesh(
        core_axis_name="c", subcore_axis_name="s", num_cores=_NC, num_subcores=_NS
    )
    k = pl.kernel(
        _gather_mean_body,
        out_type=jax.ShapeDtypeStruct((BATCH, EMB), jnp.float32),
        mesh=mesh,
        compiler_params=pltpu.CompilerParams(use_tc_tiling_on_sc=False),
        scratch_types=[
            pltpu.VMEM((_BPW, PAD_LEN), jnp.int32),
            pltpu.VMEM((2, PAD_LEN, EMB), jnp.float32),
            pltpu.VMEM((_BPW, EMB), jnp.float32),
            pltpu.SemaphoreType.DMA,
            pltpu.SemaphoreType.DMA,
        ],
    )
    return k(text, table)


_NC_PAD = 128


def _mlp_body(avg_ref, w1_ref, b1_ref, w2_ref, b2_ref, out_ref):
    h = (
        jnp.dot(avg_ref[:], w1_ref[:], preferred_element_type=jnp.float32)
        + b1_ref[:]
    )
    out_ref[:] = (
        jnp.dot(h, w2_ref[:], preferred_element_type=jnp.float32) + b2_ref[:]
    )


def _mlp(avg, W1, b1, W2, b2):
    blk = 1024
    W2p = jnp.pad(W2, ((0, 0), (0, _NC_PAD - N_CLASSES)))
    b2p = jnp.pad(b2, (0, _NC_PAD - N_CLASSES))
    out = pl.pallas_call(
        _mlp_body,
        grid=(BATCH // blk,),
        in_specs=[
            pl.BlockSpec((blk, EMB), lambda i: (i, 0)),
            pl.BlockSpec((EMB, HIDDEN), lambda i: (0, 0)),
            pl.BlockSpec((1, HIDDEN), lambda i: (0, 0)),
            pl.BlockSpec((HIDDEN, _NC_PAD), lambda i: (0, 0)),
            pl.BlockSpec((1, _NC_PAD), lambda i: (0, 0)),
        ],
        out_specs=pl.BlockSpec((blk, _NC_PAD), lambda i: (i, 0)),
        out_shape=jax.ShapeDtypeStruct((BATCH, _NC_PAD), jnp.float32),
    )(avg, W1, b1.reshape(1, HIDDEN), W2p, b2p.reshape(1, _NC_PAD))
    return out[:, :N_CLASSES]


def kernel(text, words_per_sentence, table, W1, b1, W2, b2):
    avg = _gather_mean(text, table)
    return _mlp(avg, W1, b1, W2, b2)


# speedup vs baseline: 17.0699x; 1.2240x over previous
import functools

import jax
import jax.numpy as jnp
from jax import lax
from jax.experimental import pallas as pl
from jax.experimental.pallas import tpu as pltpu
from jax.experimental.pallas import tpu_sc as plsc

VOCAB = 100000
EMB = 64
HIDDEN = 512
N_CLASSES = 100
BATCH = 4096
PAD_LEN = 200

_NC = 2
_NS = 16
_NW = _NC * _NS
_BPW = BATCH // _NW
_CH0 = 128
_CH1 = PAD_LEN - _CH0


_NBUF = 4


def _gather_mean_body(
    text_hbm, table_hbm, out_hbm, idx_v, rows_v, out_v, sem0, sem1, sem2, sem3
):
    wid = lax.axis_index("s") * _NC + lax.axis_index("c")
    base = wid * _BPW
    pltpu.sync_copy(text_hbm.at[pl.ds(base, _BPW)], idx_v)
    sems = (sem0, sem1, sem2, sem3)

    def _copies(s, b):
        sc = jnp.minimum(s, _BPW - 1)
        c0 = pltpu.make_async_copy(
            table_hbm.at[idx_v.at[sc, pl.ds(0, _CH0)]],
            rows_v.at[b, pl.ds(0, _CH0)],
            sems[b],
        )
        c1 = pltpu.make_async_copy(
            table_hbm.at[idx_v.at[sc, pl.ds(_CH0, _CH1)]],
            rows_v.at[b, pl.ds(_CH0, _CH1)],
            sems[b],
        )
        return c0, c1

    def _issue(s, b):
        for c in _copies(s, b):
            c.start()

    def _drain(s, b):
        for c in _copies(s, b):
            c.wait()

    scale = jnp.float32(1.0 / PAD_LEN)
    zeros = tuple(jnp.zeros((16,), jnp.float32) for _ in range(EMB // 16))

    _UNROLL = 4

    def _accum(s, b):
        def tokn(t, acc):
            t0 = _UNROLL * t
            for u in range(_UNROLL):
                acc = tuple(
                    acc[c] + rows_v[b, t0 + u, pl.ds(16 * c, 16)]
                    for c in range(EMB // 16)
                )
            return acc

        acc = lax.fori_loop(0, PAD_LEN // _UNROLL, tokn, zeros)
        for c in range(EMB // 16):
            out_v[s, pl.ds(16 * c, 16)] = acc[c] * scale

    for b in range(_NBUF):
        _issue(b, b)

    def ring_body(i, carry):
        s0 = _NBUF * i
        for b in range(_NBUF):
            s = s0 + b
            _drain(s, b)
            _accum(s, b)
            _issue(s + _NBUF, b)
        return carry

    lax.fori_loop(0, _BPW // _NBUF, ring_body, 0)
    for b in range(_NBUF):
        _drain(_BPW + b, b)
    pltpu.sync_copy(out_v, out_hbm.at[pl.ds(base, _BPW)])


def _gather_mean(text, table):
    mesh = plsc.VectorSubcoreMesh(
        core_axis_name="c", subcore_axis_name="s", num_cores=_NC, num_subcores=_NS
    )
    k = pl.kernel(
        _gather_mean_body,
        out_type=jax.ShapeDtypeStruct((BATCH, EMB), jnp.float32),
        mesh=mesh,
        compiler_params=pltpu.CompilerParams(use_tc_tiling_on_sc=False),
        scratch_types=[
            pltpu.VMEM((_BPW, PAD_LEN), jnp.int32),
            pltpu.VMEM((_NBUF, PAD_LEN, EMB), jnp.float32),
            pltpu.VMEM((_BPW, EMB), jnp.float32),
            pltpu.SemaphoreType.DMA,
            pltpu.SemaphoreType.DMA,
            pltpu.SemaphoreType.DMA,
            pltpu.SemaphoreType.DMA,
        ],
    )
    return k(text, table)


_NC_PAD = 128


def _mlp_body(avg_ref, w1_ref, b1_ref, w2_ref, b2_ref, out_ref):
    h = (
        jnp.dot(avg_ref[:], w1_ref[:], preferred_element_type=jnp.float32)
        + b1_ref[:]
    )
    out_ref[:] = (
        jnp.dot(h, w2_ref[:], preferred_element_type=jnp.float32) + b2_ref[:]
    )


def _mlp(avg, W1, b1, W2, b2):
    blk = 1024
    W2p = jnp.pad(W2, ((0, 0), (0, _NC_PAD - N_CLASSES)))
    b2p = jnp.pad(b2, (0, _NC_PAD - N_CLASSES))
    out = pl.pallas_call(
        _mlp_body,
        grid=(BATCH // blk,),
        in_specs=[
            pl.BlockSpec((blk, EMB), lambda i: (i, 0)),
            pl.BlockSpec((EMB, HIDDEN), lambda i: (0, 0)),
            pl.BlockSpec((1, HIDDEN), lambda i: (0, 0)),
            pl.BlockSpec((HIDDEN, _NC_PAD), lambda i: (0, 0)),
            pl.BlockSpec((1, _NC_PAD), lambda i: (0, 0)),
        ],
        out_specs=pl.BlockSpec((blk, _NC_PAD), lambda i: (i, 0)),
        out_shape=jax.ShapeDtypeStruct((BATCH, _NC_PAD), jnp.float32),
    )(avg, W1, b1.reshape(1, HIDDEN), W2p, b2p.reshape(1, _NC_PAD))
    return out[:, :N_CLASSES]


def kernel(text, words_per_sentence, table, W1, b1, W2, b2):
    avg = _gather_mean(text, table)
    return _mlp(avg, W1, b1, W2, b2)
